# baseline (speedup 1.0000x reference)
import functools

import jax
import jax.numpy as jnp
from jax import lax
from jax.experimental import pallas as pl
from jax.experimental.pallas import tpu as pltpu
from jax.experimental.pallas import tpu_sc as plsc

EPS = 1e-5
LN_BLK = 4000
W = 128
NC, NS = 2, 16
NW = NC * NS


def _ln_body(table_ref, gamma_ref, beta_ref, out_ref):
    t = table_ref[...]
    mean = jnp.mean(t, axis=1, keepdims=True)
    c = t - mean
    var = jnp.mean(c * c, axis=1, keepdims=True)
    res = c * jax.lax.rsqrt(var + EPS) * gamma_ref[...] + beta_ref[...]
    out_ref[...] = jnp.concatenate([res, jnp.zeros_like(res)], axis=1)


def _normalize_table_padded(table, gamma, beta):
    v, d = table.shape
    blk = LN_BLK
    assert v % blk == 0
    return pl.pallas_call(
        _ln_body,
        grid=(v // blk,),
        in_specs=[
            pl.BlockSpec((blk, d), lambda i: (i, 0)),
            pl.BlockSpec((1, d), lambda i: (0, 0)),
            pl.BlockSpec((1, d), lambda i: (0, 0)),
        ],
        out_specs=pl.BlockSpec((blk, 2 * d), lambda i: (i, 0)),
        out_shape=jax.ShapeDtypeStruct((v, 2 * d), jnp.float32),
    )(table, gamma.reshape(1, d), beta.reshape(1, d))


def _sc_gather(table_p, idx_flat, out_shape):
    b = idx_flat.shape[0]
    dp = table_p.shape[1]
    d = dp // 2
    assert b % (W * NW) == 0
    per_w = b // NW
    steps = per_w // W
    mesh = plsc.VectorSubcoreMesh(core_axis_name="c", subcore_axis_name="s")

    @functools.partial(
        pl.kernel,
        out_type=jax.ShapeDtypeStruct(out_shape, jnp.float32),
        mesh=mesh,
        scratch_types=[
            pltpu.VMEM((W,), jnp.int32),
            pltpu.VMEM((W, dp), jnp.float32),
            pltpu.VMEM((W, d), jnp.float32),
            pltpu.SemaphoreType.DMA,
        ],
    )
    def gather_kernel(table_hbm, i_hbm, o_hbm, idx_v, rows_v, pack_v, sem):
        o2 = o_hbm.reshape(b, d)
        wid = lax.axis_index("s") * NC + lax.axis_index("c")
        w_base = wid * per_w

        @pl.loop(0, steps)
        def _(s):
            base = w_base + s * W
            pltpu.sync_copy(i_hbm.at[pl.ds(base, W)], idx_v)
            pltpu.async_copy(table_hbm.at[idx_v], rows_v, sem).wait()

            @plsc.parallel_loop(0, W, step=8)
            def _(j8):
                for u in range(8):
                    for c in range(0, d, 16):
                        pack_v[j8 + u, pl.ds(c, 16)] = rows_v[j8 + u, pl.ds(c, 16)]

            pltpu.sync_copy(pack_v, o2.at[pl.ds(base, W)])

    return gather_kernel(table_p, idx_flat)


def kernel(x, table, gamma, beta):
    d = table.shape[1]
    table_p = _normalize_table_padded(table, gamma, beta)
    idx = x.reshape(-1).astype(jnp.int32)
    return _sc_gather(table_p, idx, x.shape + (d,))


# speedup vs baseline: 1.3893x; 1.3893x over previous
import functools

import jax
import jax.numpy as jnp
from jax import lax
from jax.experimental import pallas as pl
from jax.experimental.layout import Layout, with_layout_constraint
from jax.experimental.pallas import tpu as pltpu
from jax.experimental.pallas import tpu_sc as plsc

EPS = 1e-5
LN_BLK = 4000
W = 128
NC, NS = 2, 16
NW = NC * NS


def _ln_body(table_ref, gamma_ref, beta_ref, out_ref):
    t = table_ref[...]
    mean = jnp.mean(t, axis=1, keepdims=True)
    c = t - mean
    var = jnp.mean(c * c, axis=1, keepdims=True)
    res = c * jax.lax.rsqrt(var + EPS) * gamma_ref[...] + beta_ref[...]
    out_ref[...] = jnp.concatenate([res, jnp.zeros_like(res)], axis=1)


def _normalize_table_padded(table, gamma, beta):
    v, d = table.shape
    blk = LN_BLK
    assert v % blk == 0
    return pl.pallas_call(
        _ln_body,
        grid=(v // blk,),
        in_specs=[
            pl.BlockSpec((blk, d), lambda i: (i, 0)),
            pl.BlockSpec((1, d), lambda i: (0, 0)),
            pl.BlockSpec((1, d), lambda i: (0, 0)),
        ],
        out_specs=pl.BlockSpec((blk, 2 * d), lambda i: (i, 0)),
        out_shape=jax.ShapeDtypeStruct((v, 2 * d), jnp.float32),
    )(table, gamma.reshape(1, d), beta.reshape(1, d))


def _sc_gather(table_p, idx_flat, out_shape):
    b = idx_flat.shape[0]
    dp = table_p.shape[1]
    d = dp // 2
    assert b % (W * NW) == 0
    per_w = b // NW
    steps = per_w // W
    mesh = plsc.VectorSubcoreMesh(core_axis_name="c", subcore_axis_name="s")

    @functools.partial(
        pl.kernel,
        out_type=jax.ShapeDtypeStruct(out_shape, jnp.float32),
        mesh=mesh,
        scratch_types=[
            pltpu.VMEM((W,), jnp.int32),
            pltpu.VMEM((W, dp), jnp.float32),
            pltpu.VMEM((W, d), jnp.float32),
            pltpu.SemaphoreType.DMA,
        ],
    )
    def gather_kernel(table_hbm, i_hbm, o_hbm, idx_v, rows_v, pack_v, sem):
        o2 = o_hbm.reshape(b, d)
        wid = lax.axis_index("s") * NC + lax.axis_index("c")
        w_base = wid * per_w

        @pl.loop(0, steps)
        def _(s):
            base = w_base + s * W
            pltpu.sync_copy(i_hbm.at[pl.ds(base, W)], idx_v)
            pltpu.async_copy(table_hbm.at[idx_v], rows_v, sem).wait()

            @plsc.parallel_loop(0, W, step=8)
            def _(j8):
                for u in range(8):
                    for c in range(0, d, 16):
                        pack_v[j8 + u, pl.ds(c, 16)] = rows_v[j8 + u, pl.ds(c, 16)]

            pltpu.sync_copy(pack_v, o2.at[pl.ds(base, W)])

    return gather_kernel(table_p, idx_flat)


def kernel(x, table, gamma, beta):
    d = table.shape[1]
    table_p = _normalize_table_padded(table, gamma, beta)
    idx = x.reshape(-1).astype(jnp.int32)
    out = _sc_gather(table_p, idx, x.shape + (d,))
    return with_layout_constraint(out, Layout((0, 1, 2)))


# speedup vs baseline: 2.3576x; 1.6970x over previous
import functools

import jax
import jax.numpy as jnp
from jax import lax
from jax.experimental import pallas as pl
from jax.experimental.layout import Layout, with_layout_constraint
from jax.experimental.pallas import tpu as pltpu
from jax.experimental.pallas import tpu_sc as plsc

EPS = 1e-5
LN_BLK = 4000
W = 128
NC, NS = 2, 16
NW = NC * NS


def _ln_body(table_ref, gamma_ref, beta_ref, out_ref):
    t = table_ref[...]
    mean = jnp.mean(t, axis=1, keepdims=True)
    c = t - mean
    var = jnp.mean(c * c, axis=1, keepdims=True)
    res = c * jax.lax.rsqrt(var + EPS) * gamma_ref[...] + beta_ref[...]
    out_ref[...] = jnp.concatenate([res, jnp.zeros_like(res)], axis=1)


def _normalize_table_padded(table, gamma, beta):
    v, d = table.shape
    blk = LN_BLK
    assert v % blk == 0
    return pl.pallas_call(
        _ln_body,
        grid=(v // blk,),
        in_specs=[
            pl.BlockSpec((blk, d), lambda i: (i, 0)),
            pl.BlockSpec((1, d), lambda i: (0, 0)),
            pl.BlockSpec((1, d), lambda i: (0, 0)),
        ],
        out_specs=pl.BlockSpec((blk, 2 * d), lambda i: (i, 0)),
        out_shape=jax.ShapeDtypeStruct((v, 2 * d), jnp.float32),
    )(table, gamma.reshape(1, d), beta.reshape(1, d))


def _sc_gather(table_p, idx_flat, out_shape):
    b = idx_flat.shape[0]
    dp = table_p.shape[1]
    d = dp // 2
    assert b % (W * NW) == 0
    per_w = b // NW
    steps = per_w // W
    mesh = plsc.VectorSubcoreMesh(core_axis_name="c", subcore_axis_name="s")

    @functools.partial(
        pl.kernel,
        out_type=jax.ShapeDtypeStruct(out_shape, jnp.float32),
        mesh=mesh,
        scratch_types=[
            pltpu.VMEM((W,), jnp.int32),
            pltpu.VMEM((W,), jnp.int32),
            pltpu.VMEM((W, dp), jnp.float32),
            pltpu.VMEM((W, dp), jnp.float32),
            pltpu.VMEM((W, d), jnp.float32),
            pltpu.VMEM((W, d), jnp.float32),
            pltpu.SemaphoreType.DMA,
            pltpu.SemaphoreType.DMA,
            pltpu.SemaphoreType.DMA,
            pltpu.SemaphoreType.DMA,
        ],
    )
    def gather_kernel(table_hbm, i_hbm, o_hbm,
                      idx0, idx1, rows0, rows1, pack0, pack1, g0, g1, w0, w1):
        o2 = o_hbm.reshape(b, d)
        wid = lax.axis_index("s") * NC + lax.axis_index("c")
        w_base = wid * per_w

        def fire(s, idxb, rowsb, gsem):
            base = w_base + s * W
            pltpu.sync_copy(i_hbm.at[pl.ds(base, W)], idxb)
            pltpu.async_copy(table_hbm.at[idxb], rowsb, gsem)

        def wait_gather(idxb, rowsb, gsem):
            pltpu.make_async_copy(table_hbm.at[idxb], rowsb, gsem).wait()

        def pack(rowsb, packb):
            @pl.loop(0, W, step=8)
            def _(j8):
                for u in range(8):
                    for c in range(0, d, 16):
                        packb[j8 + u, pl.ds(c, 16)] = rowsb[j8 + u, pl.ds(c, 16)]

        def fire_wb(s, packb, wsem):
            pltpu.async_copy(packb, o2.at[pl.ds(w_base + s * W, W)], wsem)

        def wait_wb(s, packb, wsem):
            pltpu.make_async_copy(packb, o2.at[pl.ds(w_base + s * W, W)], wsem).wait()

        fire(0, idx0, rows0, g0)
        fire(1, idx1, rows1, g1)
        wait_gather(idx0, rows0, g0)
        pack(rows0, pack0)
        fire_wb(0, pack0, w0)
        fire(2, idx0, rows0, g0)
        wait_gather(idx1, rows1, g1)
        pack(rows1, pack1)
        fire_wb(1, pack1, w1)
        fire(3, idx1, rows1, g1)

        @pl.loop(4, steps, step=2)
        def _(s):
            wait_gather(idx0, rows0, g0)
            wait_wb(s - 4, pack0, w0)
            pack(rows0, pack0)
            fire_wb(s - 2, pack0, w0)
            fire(s, idx0, rows0, g0)
            wait_gather(idx1, rows1, g1)
            wait_wb(s - 3, pack1, w1)
            pack(rows1, pack1)
            fire_wb(s - 1, pack1, w1)
            fire(s + 1, idx1, rows1, g1)

        wait_gather(idx0, rows0, g0)
        wait_wb(steps - 4, pack0, w0)
        pack(rows0, pack0)
        fire_wb(steps - 2, pack0, w0)
        wait_gather(idx1, rows1, g1)
        wait_wb(steps - 3, pack1, w1)
        pack(rows1, pack1)
        fire_wb(steps - 1, pack1, w1)
        wait_wb(steps - 2, pack0, w0)
        wait_wb(steps - 1, pack1, w1)

    return gather_kernel(table_p, idx_flat)


def kernel(x, table, gamma, beta):
    d = table.shape[1]
    table_p = _normalize_table_padded(table, gamma, beta)
    idx = x.reshape(-1).astype(jnp.int32)
    out = _sc_gather(table_p, idx, x.shape + (d,))
    return with_layout_constraint(out, Layout((0, 1, 2)))
